# baseline (speedup 1.0000x reference)
import functools

import jax
import jax.numpy as jnp
from jax import lax
from jax.experimental import pallas as pl
from jax.experimental.pallas import tpu as pltpu
from jax.experimental.pallas import tpu_sc as plsc

BATCH = 1024
SEQ = 200
VOCAB = 1000000
DIM = 64
LANES = 16
NUM_CORES = 2
NUM_SUBCORES = 16
NW = NUM_CORES * NUM_SUBCORES
BT = BATCH // 128
NUNITS = SEQ * BT
UNITS_PER_W = NUNITS // NW
NBUF = 5
NOUT = UNITS_PER_W // NBUF


def _body(idx_hbm, tok_hbm, pos_hbm, out_hbm,
          idx_vs, pos_v, gbufs, obufs, gsems, wsems):
    wid = lax.axis_index("s") * NUM_CORES + lax.axis_index("c")
    ubase = wid * UNITS_PER_W

    pltpu.sync_copy(pos_hbm, pos_v)

    jvecs = [lax.iota(jnp.int32, LANES) + j * LANES for j in range(8)]

    def stage_and_fire(u, b):
        l = u // BT
        bt = u % BT
        pltpu.sync_copy(idx_hbm.at[l // 8, bt, l % 8], idx_vs[b])
        return pltpu.async_copy(tok_hbm.at[idx_vs[b]], gbufs[b], gsems[b])

    for b in range(NBUF):
        stage_and_fire(ubase + b, b)

    def outer(o, carry):
        for b in range(NBUF):
            u = ubase + o * NBUF + b
            l = u // BT
            bt = u % BT
            pltpu.make_async_copy(tok_hbm.at[idx_vs[b]], gbufs[b],
                                  gsems[b]).wait()

            @pl.when(o > 0)
            def _drain():
                for g in range(8):
                    pltpu.make_async_copy(
                        obufs[b].at[pl.ds(g * 8, 8)],
                        out_hbm.at[l, g, bt], wsems[b]).wait()

            def dbody(d, c2, _b=b, _l=l):
                pvec = plsc.load_gather(
                    pos_v, [lax.broadcast(_l, (LANES,)),
                            lax.broadcast(d, (LANES,))])
                for j in range(8):
                    gv = plsc.load_gather(
                        gbufs[_b], [jvecs[j], lax.broadcast(d, (LANES,))])
                    obufs[_b][d, pl.ds(j * LANES, LANES)] = gv + pvec
                return c2

            lax.fori_loop(0, DIM, dbody, 0, unroll=2)

            for g in range(8):
                pltpu.async_copy(obufs[b].at[pl.ds(g * 8, 8)],
                                 out_hbm.at[l, g, bt], wsems[b])

            @pl.when(o < NOUT - 1)
            def _next():
                stage_and_fire(u + NBUF, b)
        return carry

    lax.fori_loop(0, NOUT, outer, 0)

    for b in range(NBUF):
        u = ubase + (NOUT - 1) * NBUF + b
        l = u // BT
        bt = u % BT
        for g in range(8):
            pltpu.make_async_copy(obufs[b].at[pl.ds(g * 8, 8)],
                                  out_hbm.at[l, g, bt], wsems[b]).wait()


@jax.jit
def _run(idx4, tok, pos):
    mesh = plsc.VectorSubcoreMesh(core_axis_name="c", subcore_axis_name="s")
    f = functools.partial(
        pl.kernel,
        out_type=jax.ShapeDtypeStruct((SEQ, 8, BT, 8, 128), jnp.float32),
        mesh=mesh,
        scratch_types=[
            [pltpu.VMEM((128,), jnp.int32)] * NBUF,
            pltpu.VMEM((SEQ, DIM), jnp.float32),
            [pltpu.VMEM((128, DIM), jnp.float32)] * NBUF,
            [pltpu.VMEM((DIM, 128), jnp.float32)] * NBUF,
            [pltpu.SemaphoreType.DMA] * NBUF,
            [pltpu.SemaphoreType.DMA] * NBUF,
        ],
        compiler_params=pltpu.CompilerParams(use_tc_tiling_on_sc=False,
                                             needs_layout_passes=False),
    )(_body)
    return f(idx4, tok, pos)


def kernel(inputs, token_table, position_table):
    idx4 = (inputs.astype(jnp.int32)
            .reshape(BT, 128, SEQ // 8, 8)
            .transpose(2, 0, 3, 1))
    out5 = _run(idx4, token_table, position_table)
    return out5.transpose(2, 4, 0, 1, 3).reshape(BATCH, SEQ, DIM)


# speedup vs baseline: 1.0731x; 1.0731x over previous
import functools

import jax
import jax.numpy as jnp
from jax import lax
from jax.experimental import pallas as pl
from jax.experimental.pallas import tpu as pltpu
from jax.experimental.pallas import tpu_sc as plsc

BATCH = 1024
SEQ = 200
VOCAB = 1000000
DIM = 64
LANES = 16
NUM_CORES = 2
NUM_SUBCORES = 16
NW = NUM_CORES * NUM_SUBCORES
BT = BATCH // 128
NUNITS = SEQ * BT
UNITS_PER_W = NUNITS // NW
NBUF = 5
NOUT = UNITS_PER_W // NBUF


def _body(idx_hbm, tok_hbm, pos_hbm, out_hbm,
          idx_vs, pos_v, gbufs, obufs, gsems, wsems):
    wid = lax.axis_index("s") * NUM_CORES + lax.axis_index("c")
    ubase = wid * UNITS_PER_W
    lmin = lax.min(ubase // BT, SEQ - 8)
    pltpu.sync_copy(pos_hbm.at[pl.ds(lmin, 8)], pos_v)

    jvecs = [lax.iota(jnp.int32, LANES) + j * LANES for j in range(8)]

    def stage_and_fire(u, b):
        l = u // BT
        bt = u % BT
        pltpu.sync_copy(idx_hbm.at[l // 8, bt, l % 8], idx_vs[b])
        return pltpu.async_copy(tok_hbm.at[idx_vs[b]], gbufs[b], gsems[b])

    for b in range(NBUF):
        stage_and_fire(ubase + b, b)

    def outer(o, carry):
        for b in range(NBUF):
            u = ubase + o * NBUF + b
            l = u // BT
            bt = u % BT
            pltpu.make_async_copy(tok_hbm.at[idx_vs[b]], gbufs[b],
                                  gsems[b]).wait()

            @pl.when(o > 0)
            def _drain():
                for g in range(8):
                    pltpu.make_async_copy(
                        obufs[b].at[pl.ds(g * 8, 8)],
                        out_hbm.at[l, g, bt], wsems[b]).wait()

            def dbody(d, c2, _b=b, _l=l):
                pvec = plsc.load_gather(
                    pos_v, [lax.broadcast(_l - lmin, (LANES,)),
                            lax.broadcast(d, (LANES,))])
                for j in range(8):
                    gv = plsc.load_gather(
                        gbufs[_b], [jvecs[j], lax.broadcast(d, (LANES,))])
                    obufs[_b][d, pl.ds(j * LANES, LANES)] = gv + pvec
                return c2

            lax.fori_loop(0, DIM, dbody, 0, unroll=2)

            for g in range(8):
                pltpu.async_copy(obufs[b].at[pl.ds(g * 8, 8)],
                                 out_hbm.at[l, g, bt], wsems[b])

            @pl.when(o < NOUT - 1)
            def _next():
                stage_and_fire(u + NBUF, b)
        return carry

    lax.fori_loop(0, NOUT, outer, 0)

    for b in range(NBUF):
        u = ubase + (NOUT - 1) * NBUF + b
        l = u // BT
        bt = u % BT
        for g in range(8):
            pltpu.make_async_copy(obufs[b].at[pl.ds(g * 8, 8)],
                                  out_hbm.at[l, g, bt], wsems[b]).wait()


@jax.jit
def _run(idx4, tok2, pos):
    mesh = plsc.VectorSubcoreMesh(core_axis_name="c", subcore_axis_name="s")
    f = functools.partial(
        pl.kernel,
        out_type=jax.ShapeDtypeStruct((SEQ, 8, BT, 8, 128), jnp.float32),
        mesh=mesh,
        scratch_types=[
            [pltpu.VMEM((128,), jnp.int32)] * NBUF,
            pltpu.VMEM((8, DIM), jnp.float32),
            [pltpu.VMEM((128, 2 * DIM), jnp.float32)] * NBUF,
            [pltpu.VMEM((DIM, 128), jnp.float32)] * NBUF,
            [pltpu.SemaphoreType.DMA] * NBUF,
            [pltpu.SemaphoreType.DMA] * NBUF,
        ],
        compiler_params=pltpu.CompilerParams(use_tc_tiling_on_sc=False,
                                             needs_layout_passes=False),
    )(_body)
    return f(idx4, tok2, pos)


def kernel(inputs, token_table, position_table):
    idx4 = (inputs.astype(jnp.int32)
            .reshape(BT, 128, SEQ // 8, 8)
            .transpose(2, 0, 3, 1))
    tok2 = jnp.pad(token_table, ((0, 0), (0, DIM)))
    out5 = _run(idx4, tok2, position_table)
    return out5.transpose(2, 4, 0, 1, 3).reshape(BATCH, SEQ, DIM)
